# baseline (speedup 1.0000x reference)
import jax
import jax.numpy as jnp
from jax.experimental import pallas as pl
from jax.experimental.pallas import tpu as pltpu

_NC = 2
_NBJ = 25
_E = 16
_H = 32


def _reduce_kernel(x_ref, emb_ref, s1_ref, s2_ref):
    j = pl.program_id(1)
    xv = x_ref[...]
    eb = emb_ref[...]
    ki = jax.lax.broadcasted_iota(jnp.int32, (8, 128), 0)
    ci = jax.lax.broadcasted_iota(jnp.int32, (8, 128), 1)
    p = (ci // 16 == ki).astype(jnp.float32)
    xq = jax.lax.dot_general(xv, p, (((1,), (0,)), ((), ())),
                             preferred_element_type=jnp.float32)
    p1 = jnp.sum(eb * xq, axis=0, keepdims=True)
    p2 = jnp.sum((eb * eb) * (xq * xq), axis=0, keepdims=True)

    @pl.when(j == 0)
    def _init():
        s1_ref[...] = p1[None]
        s2_ref[...] = p2[None]

    @pl.when(j != 0)
    def _acc():
        s1_ref[...] = s1_ref[...] + p1[None]
        s2_ref[...] = s2_ref[...] + p2[None]


def _head_kernel(s1_ref, s2_ref, wlog_ref, blog_ref, w1_ref, b1_ref,
                 w2_ref, b2_ref, wout_ref, bout_ref, o_ref):
    t1 = s1_ref[0] + s1_ref[1]
    t2 = s2_ref[0] + s2_ref[1]
    embf = t1[:, 0:_E]
    squ = t2[:, 0:_E]
    for k in range(1, 8):
        embf = embf + t1[:, k * _E:(k + 1) * _E]
        squ = squ + t2[:, k * _E:(k + 1) * _E]
    logistic = (jnp.sum(embf * wlog_ref[...], axis=1, keepdims=True)
                + blog_ref[...])
    fm = 0.5 * (embf * embf - squ)
    dn = (((1,), (1,)), ((), ()))
    h = jnp.maximum(jax.lax.dot_general(embf, w1_ref[...], dn,
                                        preferred_element_type=jnp.float32)
                    + b1_ref[...], 0.0)
    h = jnp.maximum(jax.lax.dot_general(h, w2_ref[...], dn,
                                        preferred_element_type=jnp.float32)
                    + b2_ref[...], 0.0)
    wout = wout_ref[...]
    z = (jnp.sum(h * wout[:, 0:_H], axis=1, keepdims=True)
         + jnp.sum(fm * wout[:, _H:_H + _E], axis=1, keepdims=True)
         + logistic * wout[:, _H + _E:_H + _E + 1]
         + bout_ref[...])
    o_ref[...] = jax.nn.sigmoid(z)


def kernel(x, emb, w_log, b_log, w1, b1, w2, b2, w_out, b_out):
    f, e = emb.shape
    r = f // 8
    emb2 = emb.reshape(r, 8 * e)
    x8 = x.reshape(r, 8)
    br = r // (_NC * _NBJ)

    s1, s2 = pl.pallas_call(
        _reduce_kernel,
        grid=(_NC, _NBJ),
        in_specs=[
            pl.BlockSpec((br, 8), lambda i, j: (i * _NBJ + j, 0)),
            pl.BlockSpec((br, 128), lambda i, j: (i * _NBJ + j, 0)),
        ],
        out_specs=[
            pl.BlockSpec((1, 1, 128), lambda i, j: (i, 0, 0)),
            pl.BlockSpec((1, 1, 128), lambda i, j: (i, 0, 0)),
        ],
        out_shape=[
            jax.ShapeDtypeStruct((_NC, 1, 128), jnp.float32),
            jax.ShapeDtypeStruct((_NC, 1, 128), jnp.float32),
        ],
        compiler_params=pltpu.CompilerParams(
            dimension_semantics=("parallel", "arbitrary")),
    )(x8, emb2)

    out = pl.pallas_call(
        _head_kernel,
        out_shape=jax.ShapeDtypeStruct((1, 1), jnp.float32),
    )(s1, s2, w_log, b_log.reshape(1, 1), w1, b1.reshape(1, _H),
      w2, b2.reshape(1, _H), w_out, b_out.reshape(1, 1))
    return out.reshape(1)


# speedup vs baseline: 1.1021x; 1.1021x over previous
import jax
import jax.numpy as jnp
from jax.experimental import pallas as pl
from jax.experimental.pallas import tpu as pltpu

_NC = 2
_NBJ = 125
_E = 16
_H = 32


def _reduce_kernel(x_ref, emb_ref, s1_ref, s2_ref):
    j = pl.program_id(1)
    xr = x_ref[0]
    eb = emb_ref[...]
    dn = (((1,), (0,)), ((), ()))
    p1 = jax.lax.dot_general(xr, eb, dn,
                             preferred_element_type=jnp.float32)
    p2 = jax.lax.dot_general(xr * xr, eb * eb, dn,
                             preferred_element_type=jnp.float32)

    @pl.when(j == 0)
    def _init():
        s1_ref[...] = p1[None]
        s2_ref[...] = p2[None]

    @pl.when(j != 0)
    def _acc():
        s1_ref[...] = s1_ref[...] + p1[None]
        s2_ref[...] = s2_ref[...] + p2[None]


def _head_kernel(s1_ref, s2_ref, wlog_ref, blog_ref, w1_ref, b1_ref,
                 w2_ref, b2_ref, wout_ref, bout_ref, o_ref):
    embf = s1_ref[0] + s1_ref[1]
    squ = s2_ref[0] + s2_ref[1]
    logistic = (jnp.sum(embf * wlog_ref[...], axis=1, keepdims=True)
                + blog_ref[...])
    fm = 0.5 * (embf * embf - squ)
    dn = (((1,), (1,)), ((), ()))
    h = jnp.maximum(jax.lax.dot_general(embf, w1_ref[...], dn,
                                        preferred_element_type=jnp.float32)
                    + b1_ref[...], 0.0)
    h = jnp.maximum(jax.lax.dot_general(h, w2_ref[...], dn,
                                        preferred_element_type=jnp.float32)
                    + b2_ref[...], 0.0)
    wout = wout_ref[...]
    z = (jnp.sum(h * wout[:, 0:_H], axis=1, keepdims=True)
         + jnp.sum(fm * wout[:, _H:_H + _E], axis=1, keepdims=True)
         + logistic * wout[:, _H + _E:_H + _E + 1]
         + bout_ref[...])
    o_ref[...] = jax.nn.sigmoid(z)


def kernel(x, emb, w_log, b_log, w1, b1, w2, b2, w_out, b_out):
    f, e = emb.shape
    nb = _NC * _NBJ
    bf = f // nb
    x3 = x.reshape(nb, 1, bf)

    s1, s2 = pl.pallas_call(
        _reduce_kernel,
        grid=(_NC, _NBJ),
        in_specs=[
            pl.BlockSpec((1, 1, bf), lambda i, j: (i * _NBJ + j, 0, 0)),
            pl.BlockSpec((bf, e), lambda i, j: (i * _NBJ + j, 0)),
        ],
        out_specs=[
            pl.BlockSpec((1, 1, e), lambda i, j: (i, 0, 0)),
            pl.BlockSpec((1, 1, e), lambda i, j: (i, 0, 0)),
        ],
        out_shape=[
            jax.ShapeDtypeStruct((_NC, 1, e), jnp.float32),
            jax.ShapeDtypeStruct((_NC, 1, e), jnp.float32),
        ],
        compiler_params=pltpu.CompilerParams(
            dimension_semantics=("parallel", "arbitrary")),
    )(x3, emb)

    out = pl.pallas_call(
        _head_kernel,
        out_shape=jax.ShapeDtypeStruct((1, 1), jnp.float32),
    )(s1, s2, w_log, b_log.reshape(1, 1), w1, b1.reshape(1, _H),
      w2, b2.reshape(1, _H), w_out, b_out.reshape(1, 1))
    return out.reshape(1)
